# baseline (speedup 1.0000x reference)
import jax
import jax.numpy as jnp
from jax.experimental import pallas as pl

_C1 = 1024
_C2 = 1024
_K = 128
_TL = 1024


def _body(x_ref, w_ref, b_ref, tri_ref, o_ref):
    x = x_ref[...]
    xt = x.T
    at = jax.lax.bitcast_convert_type(xt, jnp.int32) & jnp.int32(0x7FFFFFFF)
    ones = jnp.ones((8, _C1), jnp.bfloat16)
    kf = jnp.float32(_K)

    def count(mask):
        mbf = jnp.where(mask, jnp.bfloat16(1), jnp.bfloat16(0))
        c = jax.lax.dot_general(
            ones, mbf, (((1,), (0,)), ((), ())),
            preferred_element_type=jnp.float32,
        )
        return c[0:1, :]

    hi_bits = (at >> 16).astype(jnp.int16)

    def search_hi(b, t):
        cand = t | (1 << (14 - b))
        ge = count(hi_bits >= cand.astype(jnp.int16)) >= kf
        return jnp.where(ge, cand, t)

    t_hi32 = jax.lax.fori_loop(0, 15, search_hi, jnp.zeros((1, _TL), jnp.int32))
    t_hi = t_hi32.astype(jnp.int16)

    eq_hi = hi_bits == t_hi
    gt_hi = hi_bits > t_hi
    n_gt_hi = count(gt_hi)
    kf2 = kf - n_gt_hi
    lo_bits = ((at & 0xFFFF) - 32768).astype(jnp.int16)
    key16 = jnp.where(eq_hi, lo_bits, jnp.int16(-32768))

    def search_lo(b, tu):
        cand = tu | (1 << (15 - b))
        cand_s = (cand - 32768).astype(jnp.int16)
        ge = count(key16 >= cand_s) >= kf2
        return jnp.where(ge, cand, tu)

    tu = jax.lax.fori_loop(0, 16, search_lo, jnp.zeros((1, _TL), jnp.int32))
    t_lo = (tu - 32768).astype(jnp.int16)

    gt = gt_hi | (key16 > t_lo)
    eq = eq_hi & (key16 == t_lo)
    needed = kf - count(gt)
    eqb = eq.astype(jnp.bfloat16)
    prefix = jax.lax.dot_general(
        tri_ref[...], eqb, (((1,), (0,)), ((), ())),
        preferred_element_type=jnp.float32,
    )
    mask = gt | (eq & (prefix <= needed))
    xmt = jnp.where(mask, xt, 0.0).astype(jnp.bfloat16)
    out = jax.lax.dot_general(
        xmt, w_ref[...], (((0,), (0,)), ((), ())),
        preferred_element_type=jnp.float32,
    )
    o_ref[...] = out + b_ref[...]


def kernel(x, weight, bias):
    b, l, c1 = x.shape
    x2 = x.reshape(b * l, c1)
    r = jax.lax.broadcasted_iota(jnp.int32, (_C1, _C1), 0)
    c = jax.lax.broadcasted_iota(jnp.int32, (_C1, _C1), 1)
    tri = (r >= c).astype(jnp.bfloat16)
    out = pl.pallas_call(
        _body,
        grid=((b * l) // _TL,),
        in_specs=[
            pl.BlockSpec((_TL, _C1), lambda i: (i, 0)),
            pl.BlockSpec((_C1, _C2), lambda i: (0, 0)),
            pl.BlockSpec((1, _C2), lambda i: (0, 0)),
            pl.BlockSpec((_C1, _C1), lambda i: (0, 0)),
        ],
        out_specs=pl.BlockSpec((_TL, _C2), lambda i: (i, 0)),
        out_shape=jax.ShapeDtypeStruct((b * l, _C2), jnp.float32),
    )(x2, weight.astype(jnp.bfloat16), bias.reshape(1, _C2), tri)
    return out.reshape(b, l, _C2)


# speedup vs baseline: 1.3502x; 1.3502x over previous
import jax
import jax.numpy as jnp
from jax.experimental import pallas as pl

_C1 = 1024
_C2 = 1024
_K = 128
_TL = 1024
_H = _TL // 2


def _body(x_ref, w_ref, b_ref, tri_ref, o_ref):
    x = x_ref[...]
    xt = x.T
    at = jax.lax.bitcast_convert_type(xt, jnp.int32) & jnp.int32(0x7FFFFFFF)
    ones = jnp.ones((8, _C1), jnp.bfloat16)
    kf = jnp.float32(_K)

    def count(mask):
        mbf = jnp.where(mask, jnp.bfloat16(1), jnp.bfloat16(0))
        c = jax.lax.dot_general(
            ones, mbf, (((1,), (0,)), ((), ())),
            preferred_element_type=jnp.float32,
        )
        return c[0:1, :]

    hi_bits = (at >> 16).astype(jnp.int16)
    hi_a = hi_bits[:, :_H]
    hi_b = hi_bits[:, _H:]

    z = jnp.zeros((1, _H), jnp.int32)
    ta, tb = z, z
    for b in range(15):
        bit = 1 << (14 - b)
        ca, cb = ta | bit, tb | bit
        ga = count(hi_a >= ca.astype(jnp.int16)) >= kf
        gb = count(hi_b >= cb.astype(jnp.int16)) >= kf
        ta, tb = jnp.where(ga, ca, ta), jnp.where(gb, cb, tb)
    t_hi = jnp.concatenate([ta, tb], axis=1).astype(jnp.int16)

    eq_hi = hi_bits == t_hi
    gt_hi = hi_bits > t_hi
    n_gt_hi = count(gt_hi)
    kf2 = kf - n_gt_hi
    kf2a, kf2b = kf2[:, :_H], kf2[:, _H:]
    lo_bits = ((at & 0xFFFF) - 32768).astype(jnp.int16)
    key16 = jnp.where(eq_hi, lo_bits, jnp.int16(-32768))
    key_a = key16[:, :_H]
    key_b = key16[:, _H:]

    tua, tub = z, z
    for b in range(16):
        bit = 1 << (15 - b)
        ca, cb = tua | bit, tub | bit
        ga = count(key_a >= (ca - 32768).astype(jnp.int16)) >= kf2a
        gb = count(key_b >= (cb - 32768).astype(jnp.int16)) >= kf2b
        tua, tub = jnp.where(ga, ca, tua), jnp.where(gb, cb, tub)
    t_lo = (jnp.concatenate([tua, tub], axis=1) - 32768).astype(jnp.int16)

    gt = gt_hi | (key16 > t_lo)
    eq = eq_hi & (key16 == t_lo)
    needed = kf - count(gt)
    eqb = jnp.where(eq, jnp.bfloat16(1), jnp.bfloat16(0))
    prefix = jax.lax.dot_general(
        tri_ref[...], eqb, (((1,), (0,)), ((), ())),
        preferred_element_type=jnp.float32,
    )
    mask = gt | (eq & (prefix <= needed))
    xmt = jnp.where(mask, xt, 0.0)
    out = jax.lax.dot_general(
        xmt, w_ref[...], (((0,), (0,)), ((), ())),
        preferred_element_type=jnp.float32,
    )
    o_ref[...] = out + b_ref[...]


def kernel(x, weight, bias):
    b, l, c1 = x.shape
    x2 = x.reshape(b * l, c1)
    r = jax.lax.broadcasted_iota(jnp.int32, (_C1, _C1), 0)
    c = jax.lax.broadcasted_iota(jnp.int32, (_C1, _C1), 1)
    tri = (r >= c).astype(jnp.bfloat16)
    out = pl.pallas_call(
        _body,
        grid=((b * l) // _TL,),
        in_specs=[
            pl.BlockSpec((_TL, _C1), lambda i: (i, 0)),
            pl.BlockSpec((_C1, _C2), lambda i: (0, 0)),
            pl.BlockSpec((1, _C2), lambda i: (0, 0)),
            pl.BlockSpec((_C1, _C1), lambda i: (0, 0)),
        ],
        out_specs=pl.BlockSpec((_TL, _C2), lambda i: (i, 0)),
        out_shape=jax.ShapeDtypeStruct((b * l, _C2), jnp.float32),
    )(x2, weight, bias.reshape(1, _C2), tri)
    return out.reshape(b, l, _C2)
